# initial kernel scaffold (unmeasured)
import jax
import jax.numpy as jnp
from jax import lax
from jax.experimental import pallas as pl
from jax.experimental.pallas import tpu as pltpu

N_DEV = 16


def kernel(x, w_mat):
    m_per, k = x.shape
    _, n = w_mat.shape
    n_per = n // N_DEV

    def body(x_ref, w_ref, out_ref, y_ref, amax_ref, gather_ref,
             data_send_sems, data_recv_sems, amax_send_sems, amax_recv_sems):
        my = lax.axis_index("i")

        barrier_sem = pltpu.get_barrier_semaphore()
        for j in range(1, N_DEV):
            t = lax.rem(my + j, N_DEV)
            pl.semaphore_signal(
                barrier_sem, inc=1,
                device_id=(t,), device_id_type=pl.DeviceIdType.MESH,
            )
        pl.semaphore_wait(barrier_sem, N_DEV - 1)

        for t in range(N_DEV):
            y_ref[t] = jnp.dot(
                x_ref[...], w_ref[:, t * n_per:(t + 1) * n_per],
                preferred_element_type=jnp.float32,
            )
        local_amax = jnp.max(jnp.abs(y_ref[...]))
        amax_ref[...] = jnp.broadcast_to(local_amax, (1, 128))
        gather_ref[pl.ds(my, 1), :] = amax_ref[...]

        out_ref[pl.ds(my * m_per, m_per), :] = y_ref[my]

        data_rdmas = []
        amax_rdmas = []
        for j in range(1, N_DEV):
            t = lax.rem(my + j, N_DEV)
            rdma = pltpu.make_async_remote_copy(
                src_ref=y_ref.at[t],
                dst_ref=out_ref.at[pl.ds(my * m_per, m_per), :],
                send_sem=data_send_sems.at[j],
                recv_sem=data_recv_sems.at[j],
                device_id=(t,),
                device_id_type=pl.DeviceIdType.MESH,
            )
            rdma.start()
            data_rdmas.append(rdma)
            a_rdma = pltpu.make_async_remote_copy(
                src_ref=amax_ref,
                dst_ref=gather_ref.at[pl.ds(my, 1), :],
                send_sem=amax_send_sems.at[j],
                recv_sem=amax_recv_sems.at[j],
                device_id=(t,),
                device_id_type=pl.DeviceIdType.MESH,
            )
            a_rdma.start()
            amax_rdmas.append(a_rdma)

        for rdma in data_rdmas:
            rdma.wait()
        for a_rdma in amax_rdmas:
            a_rdma.wait()

        scale = jnp.max(gather_ref[...]) / 127.0
        q = jnp.clip(jnp.round(out_ref[...] / scale), -127.0, 127.0)
        out_ref[...] = q * scale

    return pl.pallas_call(
        body,
        out_shape=jax.ShapeDtypeStruct((N_DEV * m_per, n_per), jnp.float32),
        in_specs=[
            pl.BlockSpec(memory_space=pltpu.VMEM),
            pl.BlockSpec(memory_space=pltpu.VMEM),
        ],
        out_specs=pl.BlockSpec(memory_space=pltpu.VMEM),
        scratch_shapes=[
            pltpu.VMEM((N_DEV, m_per, n_per), jnp.float32),
            pltpu.VMEM((1, 128), jnp.float32),
            pltpu.VMEM((N_DEV, 128), jnp.float32),
            pltpu.SemaphoreType.DMA((N_DEV,)),
            pltpu.SemaphoreType.DMA((N_DEV,)),
            pltpu.SemaphoreType.DMA((N_DEV,)),
            pltpu.SemaphoreType.DMA((N_DEV,)),
        ],
        compiler_params=pltpu.CompilerParams(collective_id=0),
    )(x, w_mat)


# baseline (device time: 51978 ns/iter reference)
import jax
import jax.numpy as jnp
from jax import lax
from jax.experimental import pallas as pl
from jax.experimental.pallas import tpu as pltpu

N_DEV = 16


def kernel(x, w_mat):
    m_per, k = x.shape
    _, n = w_mat.shape
    n_per = n // N_DEV

    def body(x_ref, w_ref, out_ref, y_ref, amax_ref, gather_ref,
             data_send_sems, data_recv_sems, amax_send_sems, amax_recv_sems):
        my = lax.axis_index("i")

        barrier_sem = pltpu.get_barrier_semaphore()
        for j in range(1, N_DEV):
            t = lax.rem(my + j, N_DEV)
            pl.semaphore_signal(
                barrier_sem, inc=1,
                device_id=(t,), device_id_type=pl.DeviceIdType.MESH,
            )
        pl.semaphore_wait(barrier_sem, N_DEV - 1)

        for t in range(N_DEV):
            y_ref[t] = jnp.dot(
                x_ref[...], w_ref[:, t * n_per:(t + 1) * n_per],
                preferred_element_type=jnp.float32,
            )
        local_amax = jnp.max(jnp.abs(y_ref[...]))
        amax_ref[...] = jnp.broadcast_to(local_amax, (1, 128))
        gather_ref[pl.ds(my, 1), :] = amax_ref[...]

        out_ref[pl.ds(my * m_per, m_per), :] = y_ref[my]

        data_rdmas = []
        amax_rdmas = []
        for j in range(1, N_DEV):
            t = lax.rem(my + j, N_DEV)
            rdma = pltpu.make_async_remote_copy(
                src_ref=y_ref.at[t],
                dst_ref=out_ref.at[pl.ds(my * m_per, m_per), :],
                send_sem=data_send_sems.at[j],
                recv_sem=data_recv_sems.at[j],
                device_id=(t,),
                device_id_type=pl.DeviceIdType.MESH,
            )
            rdma.start()
            data_rdmas.append(rdma)
            a_rdma = pltpu.make_async_remote_copy(
                src_ref=amax_ref,
                dst_ref=gather_ref.at[pl.ds(my, 1), :],
                send_sem=amax_send_sems.at[j],
                recv_sem=amax_recv_sems.at[j],
                device_id=(t,),
                device_id_type=pl.DeviceIdType.MESH,
            )
            a_rdma.start()
            amax_rdmas.append(a_rdma)

        for rdma in data_rdmas:
            rdma.wait()
        for a_rdma in amax_rdmas:
            a_rdma.wait()

        scale = jnp.max(gather_ref[...]) / 127.0
        q = jnp.clip(jnp.round(out_ref[...] / scale), -127.0, 127.0)
        out_ref[...] = q * scale

    return pl.pallas_call(
        body,
        out_shape=jax.ShapeDtypeStruct((N_DEV * m_per, n_per), jnp.float32),
        in_specs=[
            pl.BlockSpec(memory_space=pltpu.VMEM),
            pl.BlockSpec(memory_space=pltpu.VMEM),
        ],
        out_specs=pl.BlockSpec(memory_space=pltpu.VMEM),
        scratch_shapes=[
            pltpu.VMEM((N_DEV, m_per, n_per), jnp.float32),
            pltpu.VMEM((1, 128), jnp.float32),
            pltpu.VMEM((N_DEV, 128), jnp.float32),
            pltpu.SemaphoreType.DMA((N_DEV,)),
            pltpu.SemaphoreType.DMA((N_DEV,)),
            pltpu.SemaphoreType.DMA((N_DEV,)),
            pltpu.SemaphoreType.DMA((N_DEV,)),
        ],
        compiler_params=pltpu.CompilerParams(
            collective_id=0, vmem_limit_bytes=100 * 1024 * 1024,
        ),
    )(x, w_mat)
